# baseline (speedup 1.0000x reference)
import functools

import jax
import jax.numpy as jnp
from jax.experimental import pallas as pl

_BF = jnp.float32


def _dot(a, b, out=jnp.float32):
    return jax.lax.dot(a, b, preferred_element_type=out)


def _fused_body(xa_ref, w1a_ref, *refs, ndim, inter):
    out_ref = refs[-1]
    h1 = jnp.maximum(_dot(xa_ref[...], w1a_ref[...]), 0)
    h3s = []
    for d in range(ndim):
        w2, b2, w3, b3 = refs[4 * d:4 * d + 4]
        h = h1[:, d * inter:(d + 1) * inter]
        h = jnp.maximum(_dot(h, w2[...]) + b2[...], 0)
        h = jnp.maximum(_dot(h, w3[...]) + b3[...], 0)
        h3s.append(h)
    w4s, bc, wr2, br2, wr3, br3, wr4, br4 = refs[4 * ndim:4 * ndim + 8]
    hcat = jnp.concatenate(h3s, axis=1)
    h = jnp.maximum(_dot(hcat, w4s[...]) + bc[...], 0)
    h = jnp.maximum(_dot(h, wr2[...]) + br2[...], 0)
    h = jnp.maximum(_dot(h, wr3[...]) + br3[...], 0)
    out_ref[...] = _dot(h, wr4[...], out=jnp.float32) + br4[...]


def kernel(x, nets, rho_params):
    n, ndim = x.shape
    inter = nets[0][-1][0].shape[1]
    zdim = rho_params[-1][0].shape[1]

    wr1, br1 = rho_params[0]

    w1a = jnp.zeros((ndim + 1, ndim * inter), jnp.float32)
    for d, net in enumerate(nets):
        w1a = w1a.at[d, d * inter:(d + 1) * inter].set(net[0][0][0])
        w1a = w1a.at[ndim, d * inter:(d + 1) * inter].set(net[0][1])
    xa = jnp.concatenate([x, jnp.ones((n, 1), x.dtype)], axis=1)

    args = []
    bc_terms = br1
    for net in nets:
        (w2, b2), (w3, b3) = net[1], net[2]
        args += [w2.astype(_BF), b2[None, :].astype(_BF),
                 w3.astype(_BF), b3[None, :].astype(_BF)]
    w4s = jnp.concatenate([net[3][0] @ wr1 for net in nets], axis=0)
    for net in nets:
        bc_terms = bc_terms + net[3][1] @ wr1
    args.append(w4s.astype(_BF))
    args.append(bc_terms[None, :].astype(_BF))
    for (w, b) in rho_params[1:-1]:
        args += [w.astype(_BF), b[None, :].astype(_BF)]
    args += [rho_params[-1][0].astype(_BF), rho_params[-1][1][None, :]]

    blk = 1024
    n_pad = ((n + blk - 1) // blk) * blk
    if n_pad != n:
        xa = jnp.pad(xa, ((0, n_pad - n), (0, 0)))
    xa = xa.astype(_BF)
    w1a = w1a.astype(_BF)

    full = lambda a: pl.BlockSpec(a.shape, lambda i: (0,) * a.ndim)
    out = pl.pallas_call(
        functools.partial(_fused_body, ndim=ndim, inter=inter),
        grid=(n_pad // blk,),
        in_specs=[pl.BlockSpec((blk, ndim + 1), lambda i: (i, 0)), full(w1a)]
                 + [full(a) for a in args],
        out_specs=pl.BlockSpec((blk, zdim), lambda i: (i, 0)),
        out_shape=jax.ShapeDtypeStruct((n_pad, zdim), jnp.float32),
    )(xa, w1a, *args)
    return out[:n] if n_pad != n else out


# speedup vs baseline: 1.4506x; 1.4506x over previous
import functools

import jax
import jax.numpy as jnp
from jax.experimental import pallas as pl


def _fused_body(x_ref, w1_ref, b1_ref, w2_ref, b2_ref, w3_ref, b3_ref,
                w4c_ref, bc_ref, wr2_ref, br2_ref, wr3_ref, br3_ref,
                wr4_ref, br4_ref, out_ref, *, ndim):
    x = x_ref[...]
    acc = None
    for d in range(ndim):
        col = x[:, d:d + 1]
        h = jnp.maximum(col * w1_ref[d:d + 1, :] + b1_ref[d:d + 1, :], 0.0)
        h = jnp.maximum(
            jnp.dot(h, w2_ref[d], preferred_element_type=jnp.float32)
            + b2_ref[d:d + 1, :], 0.0)
        h = jnp.maximum(
            jnp.dot(h, w3_ref[d], preferred_element_type=jnp.float32)
            + b3_ref[d:d + 1, :], 0.0)
        g = jnp.dot(h, w4c_ref[d], preferred_element_type=jnp.float32)
        acc = g if acc is None else acc + g
    h = jnp.maximum(acc + bc_ref[...], 0.0)
    h = jnp.maximum(
        jnp.dot(h, wr2_ref[...], preferred_element_type=jnp.float32)
        + br2_ref[...], 0.0)
    h = jnp.maximum(
        jnp.dot(h, wr3_ref[...], preferred_element_type=jnp.float32)
        + br3_ref[...], 0.0)
    out_ref[...] = (
        jnp.dot(h, wr4_ref[...], preferred_element_type=jnp.float32)
        + br4_ref[...])


def kernel(x, nets, rho_params):
    n, ndim = x.shape
    zdim = rho_params[-1][0].shape[1]

    w1 = jnp.concatenate([net[0][0] for net in nets], axis=0)
    b1 = jnp.stack([net[0][1] for net in nets], axis=0)
    w2 = jnp.stack([net[1][0] for net in nets], axis=0)
    b2 = jnp.stack([net[1][1] for net in nets], axis=0)
    w3 = jnp.stack([net[2][0] for net in nets], axis=0)
    b3 = jnp.stack([net[2][1] for net in nets], axis=0)
    w4 = jnp.stack([net[3][0] for net in nets], axis=0)
    b4 = jnp.stack([net[3][1] for net in nets], axis=0)

    wr1, br1 = rho_params[0]
    wr2, br2 = rho_params[1]
    wr3, br3 = rho_params[2]
    wr4, br4 = rho_params[3]

    w4c = jnp.einsum('dij,jk->dik', w4, wr1)
    bc = (jnp.sum(b4, axis=0) @ wr1 + br1)[None, :]

    blk = 1024
    n_pad = ((n + blk - 1) // blk) * blk
    xp = x if n_pad == n else jnp.pad(x, ((0, n_pad - n), (0, 0)))

    full = lambda a: pl.BlockSpec(a.shape, lambda i: (0,) * a.ndim)
    args = (w1, b1, w2, b2, w3, b3, w4c, bc,
            wr2, br2[None, :], wr3, br3[None, :], wr4, br4[None, :])

    out = pl.pallas_call(
        functools.partial(_fused_body, ndim=ndim),
        grid=(n_pad // blk,),
        in_specs=[pl.BlockSpec((blk, ndim), lambda i: (i, 0))]
                 + [full(a) for a in args],
        out_specs=pl.BlockSpec((blk, zdim), lambda i: (i, 0)),
        out_shape=jax.ShapeDtypeStruct((n_pad, zdim), jnp.float32),
    )(xp, *args)
    return out[:n] if n_pad != n else out


# speedup vs baseline: 1.5530x; 1.0706x over previous
import functools

import jax
import jax.numpy as jnp
from jax.experimental import pallas as pl


def _fused_body(x_ref, *refs, ndim, inter):
    out_ref = refs[-1]
    w4s, bc, wr2, br2, wr3, br3, wr4, br4 = refs[6 * ndim:6 * ndim + 8]
    x = x_ref[...]
    acc = None
    for d in range(ndim):
        w1, b1, w2, b2, w3, b3 = refs[6 * d:6 * d + 6]
        col = x[:, d:d + 1]
        h = jnp.maximum(col * w1[...] + b1[...], 0.0)
        h = jnp.maximum(
            jnp.dot(h, w2[...], preferred_element_type=jnp.float32)
            + b2[...], 0.0)
        h = jnp.maximum(
            jnp.dot(h, w3[...], preferred_element_type=jnp.float32)
            + b3[...], 0.0)
        g = jnp.dot(h, w4s[d * inter:(d + 1) * inter, :],
                    preferred_element_type=jnp.float32)
        acc = g if acc is None else acc + g
    h = jnp.maximum(acc + bc[...], 0.0)
    h = jnp.maximum(
        jnp.dot(h, wr2[...], preferred_element_type=jnp.float32)
        + br2[...], 0.0)
    h = jnp.maximum(
        jnp.dot(h, wr3[...], preferred_element_type=jnp.float32)
        + br3[...], 0.0)
    out_ref[...] = (
        jnp.dot(h, wr4[...], preferred_element_type=jnp.float32)
        + br4[...])


def kernel(x, nets, rho_params):
    n, ndim = x.shape
    inter = nets[0][-1][0].shape[1]
    zdim = rho_params[-1][0].shape[1]

    wr1, br1 = rho_params[0]

    args = []
    for net in nets:
        (w1, b1), (w2, b2), (w3, b3) = net[0], net[1], net[2]
        args += [w1, b1[None, :], w2, b2[None, :], w3, b3[None, :]]
    w4s = jnp.concatenate([net[3][0] for net in nets], axis=0) @ wr1
    bc = br1
    for net in nets:
        bc = bc + net[3][1] @ wr1
    args += [w4s, bc[None, :]]
    for (w, b) in rho_params[1:]:
        args += [w, b[None, :]]

    blk = 1024
    n_pad = ((n + blk - 1) // blk) * blk
    xp = x if n_pad == n else jnp.pad(x, ((0, n_pad - n), (0, 0)))

    full = lambda a: pl.BlockSpec(a.shape, lambda i: (0,) * a.ndim)
    out = pl.pallas_call(
        functools.partial(_fused_body, ndim=ndim, inter=inter),
        grid=(n_pad // blk,),
        in_specs=[pl.BlockSpec((blk, ndim), lambda i: (i, 0))]
                 + [full(a) for a in args],
        out_specs=pl.BlockSpec((blk, zdim), lambda i: (i, 0)),
        out_shape=jax.ShapeDtypeStruct((n_pad, zdim), jnp.float32),
    )(xp, *args)
    return out[:n] if n_pad != n else out


# speedup vs baseline: 1.6667x; 1.0732x over previous
import functools

import jax
import jax.numpy as jnp
from jax.experimental import pallas as pl


def _fused_body(x_ref, *refs, ndim, inter):
    out_ref = refs[-1]
    w4s, bc, wr2, br2, wr3, br3, wr4, br4 = refs[6 * ndim:6 * ndim + 8]
    x = x_ref[...]
    acc = None
    for d in range(ndim):
        w1, b1, w2, b2, w3, b3 = refs[6 * d:6 * d + 6]
        col = x[:, d:d + 1]
        h = jnp.maximum(col * w1[...] + b1[...], 0.0)
        h = jnp.maximum(
            jnp.dot(h, w2[...], preferred_element_type=jnp.float32)
            + b2[...], 0.0)
        h = jnp.maximum(
            jnp.dot(h, w3[...], preferred_element_type=jnp.float32)
            + b3[...], 0.0)
        g = jnp.dot(h, w4s[d * inter:(d + 1) * inter, :],
                    preferred_element_type=jnp.float32)
        acc = g if acc is None else acc + g
    h = jnp.maximum(acc + bc[...], 0.0)
    h = jnp.maximum(
        jnp.dot(h, wr2[...], preferred_element_type=jnp.float32)
        + br2[...], 0.0)
    h = jnp.maximum(
        jnp.dot(h, wr3[...], preferred_element_type=jnp.float32)
        + br3[...], 0.0)
    out_ref[...] = (
        jnp.dot(h, wr4[...], preferred_element_type=jnp.float32)
        + br4[...])


def kernel(x, nets, rho_params):
    n, ndim = x.shape
    inter = nets[0][-1][0].shape[1]
    zdim = rho_params[-1][0].shape[1]

    wr1, br1 = rho_params[0]

    args = []
    for net in nets:
        (w1, b1), (w2, b2), (w3, b3) = net[0], net[1], net[2]
        args += [w1, b1[None, :], w2, b2[None, :], w3, b3[None, :]]
    w4s = jnp.concatenate([net[3][0] for net in nets], axis=0) @ wr1
    bc = br1
    for net in nets:
        bc = bc + net[3][1] @ wr1
    args += [w4s, bc[None, :]]
    for (w, b) in rho_params[1:]:
        args += [w, b[None, :]]

    blk = 2048
    n_pad = ((n + blk - 1) // blk) * blk
    xp = x if n_pad == n else jnp.pad(x, ((0, n_pad - n), (0, 0)))

    full = lambda a: pl.BlockSpec(a.shape, lambda i: (0,) * a.ndim)
    out = pl.pallas_call(
        functools.partial(_fused_body, ndim=ndim, inter=inter),
        grid=(n_pad // blk,),
        in_specs=[pl.BlockSpec((blk, ndim), lambda i: (i, 0))]
                 + [full(a) for a in args],
        out_specs=pl.BlockSpec((blk, zdim), lambda i: (i, 0)),
        out_shape=jax.ShapeDtypeStruct((n_pad, zdim), jnp.float32),
    )(xp, *args)
    return out[:n] if n_pad != n else out


# speedup vs baseline: 1.6858x; 1.0114x over previous
import functools

import jax
import jax.numpy as jnp
from jax.experimental import pallas as pl


def _fused_body(x_ref, *refs, ndim, inter):
    out_ref = refs[-1]
    w4s, bc, wr2, br2, wr3, br3, wr4, br4 = refs[6 * ndim:6 * ndim + 8]
    x = x_ref[...]
    acc = None
    for d in range(ndim):
        w1, b1, w2, b2, w3, b3 = refs[6 * d:6 * d + 6]
        col = x[:, d:d + 1]
        h = jnp.maximum(col * w1[...] + b1[...], 0.0)
        h = jnp.maximum(
            jnp.dot(h, w2[...], preferred_element_type=jnp.float32)
            + b2[...], 0.0)
        h = jnp.maximum(
            jnp.dot(h, w3[...], preferred_element_type=jnp.float32)
            + b3[...], 0.0)
        g = jnp.dot(h, w4s[d * inter:(d + 1) * inter, :],
                    preferred_element_type=jnp.float32)
        acc = g if acc is None else acc + g
    h = jnp.maximum(acc + bc[...], 0.0)
    h = jnp.maximum(
        jnp.dot(h, wr2[...], preferred_element_type=jnp.float32)
        + br2[...], 0.0)
    h = jnp.maximum(
        jnp.dot(h, wr3[...], preferred_element_type=jnp.float32)
        + br3[...], 0.0)
    out_ref[...] = (
        jnp.dot(h, wr4[...], preferred_element_type=jnp.float32)
        + br4[...])


def kernel(x, nets, rho_params):
    n, ndim = x.shape
    inter = nets[0][-1][0].shape[1]
    zdim = rho_params[-1][0].shape[1]

    wr1, br1 = rho_params[0]

    args = []
    for net in nets:
        (w1, b1), (w2, b2), (w3, b3) = net[0], net[1], net[2]
        args += [w1, b1[None, :], w2, b2[None, :], w3, b3[None, :]]
    w4s = jnp.concatenate([net[3][0] for net in nets], axis=0) @ wr1
    bc = br1
    for net in nets:
        bc = bc + net[3][1] @ wr1
    args += [w4s, bc[None, :]]
    for (w, b) in rho_params[1:]:
        args += [w, b[None, :]]

    blk = 4096
    n_pad = ((n + blk - 1) // blk) * blk
    xp = x if n_pad == n else jnp.pad(x, ((0, n_pad - n), (0, 0)))

    full = lambda a: pl.BlockSpec(a.shape, lambda i: (0,) * a.ndim)
    out = pl.pallas_call(
        functools.partial(_fused_body, ndim=ndim, inter=inter),
        grid=(n_pad // blk,),
        in_specs=[pl.BlockSpec((blk, ndim), lambda i: (i, 0))]
                 + [full(a) for a in args],
        out_specs=pl.BlockSpec((blk, zdim), lambda i: (i, 0)),
        out_shape=jax.ShapeDtypeStruct((n_pad, zdim), jnp.float32),
    )(xp, *args)
    return out[:n] if n_pad != n else out


# speedup vs baseline: 1.6899x; 1.0024x over previous
import functools

import jax
import jax.numpy as jnp
from jax.experimental import pallas as pl


def _fused_body(x_ref, *refs, ndim, inter):
    out_ref = refs[-1]
    w4s, bc, wr2, br2, wr3, br3, wr4, br4 = refs[6 * ndim:6 * ndim + 8]
    x = x_ref[...]
    acc = None
    for d in range(ndim):
        w1, b1, w2, b2, w3, b3 = refs[6 * d:6 * d + 6]
        col = x[:, d:d + 1]
        h = jnp.maximum(col * w1[...] + b1[...], 0.0)
        h = jnp.maximum(
            jnp.dot(h, w2[...], preferred_element_type=jnp.float32)
            + b2[...], 0.0)
        h = jnp.maximum(
            jnp.dot(h, w3[...], preferred_element_type=jnp.float32)
            + b3[...], 0.0)
        g = jnp.dot(h, w4s[d * inter:(d + 1) * inter, :],
                    preferred_element_type=jnp.float32)
        acc = g if acc is None else acc + g
    h = jnp.maximum(acc + bc[...], 0.0)
    h = jnp.maximum(
        jnp.dot(h, wr2[...], preferred_element_type=jnp.float32)
        + br2[...], 0.0)
    h = jnp.maximum(
        jnp.dot(h, wr3[...], preferred_element_type=jnp.float32)
        + br3[...], 0.0)
    out_ref[...] = (
        jnp.dot(h, wr4[...], preferred_element_type=jnp.float32)
        + br4[...])


def kernel(x, nets, rho_params):
    n, ndim = x.shape
    inter = nets[0][-1][0].shape[1]
    zdim = rho_params[-1][0].shape[1]

    wr1, br1 = rho_params[0]

    args = []
    for net in nets:
        (w1, b1), (w2, b2), (w3, b3) = net[0], net[1], net[2]
        args += [w1, b1, w2, b2, w3, b3]
    w4s = jnp.concatenate([net[3][0] for net in nets], axis=0) @ wr1
    bc = br1
    for net in nets:
        bc = bc + net[3][1] @ wr1
    args += [w4s, bc]
    for (w, b) in rho_params[1:]:
        args += [w, b]

    blk = 4096
    n_pad = ((n + blk - 1) // blk) * blk
    xp = x if n_pad == n else jnp.pad(x, ((0, n_pad - n), (0, 0)))

    full = lambda a: pl.BlockSpec(a.shape, lambda i: (0,) * a.ndim)
    out = pl.pallas_call(
        functools.partial(_fused_body, ndim=ndim, inter=inter),
        grid=(n_pad // blk,),
        in_specs=[pl.BlockSpec((blk, ndim), lambda i: (i, 0))]
                 + [full(a) for a in args],
        out_specs=pl.BlockSpec((blk, zdim), lambda i: (i, 0)),
        out_shape=jax.ShapeDtypeStruct((n_pad, zdim), jnp.float32),
    )(xp, *args)
    return out[:n] if n_pad != n else out


# speedup vs baseline: 1.9479x; 1.1527x over previous
import functools

import jax
import jax.numpy as jnp
from jax.experimental import pallas as pl
from jax.experimental.pallas import tpu as pltpu


def _dot(a, b):
    return jnp.dot(a, b, preferred_element_type=jnp.float32)


def _fused_body(x_ref, *refs, ndim, inter):
    wr1, br1, wr2, br2, wr3, br3, wr4, br4 = refs[8 * ndim:8 * ndim + 8]
    out_ref, w4s, bc = refs[8 * ndim + 8:]

    @pl.when(pl.program_id(0) == 0)
    def _compose():
        b4sum = None
        for d in range(ndim):
            w4, b4 = refs[8 * d + 6], refs[8 * d + 7]
            w4s[d * inter:(d + 1) * inter, :] = _dot(w4[...], wr1[...])
            b4sum = b4[...] if b4sum is None else b4sum + b4[...]
        bc[...] = _dot(b4sum[None, :], wr1[...]) + br1[...][None, :]

    x = x_ref[...]
    acc = None
    for d in range(ndim):
        w1, b1, w2, b2, w3, b3 = refs[8 * d:8 * d + 6]
        col = x[:, d:d + 1]
        h = jnp.maximum(col * w1[...] + b1[...], 0.0)
        h = jnp.maximum(_dot(h, w2[...]) + b2[...], 0.0)
        h = jnp.maximum(_dot(h, w3[...]) + b3[...], 0.0)
        g = _dot(h, w4s[d * inter:(d + 1) * inter, :])
        acc = g if acc is None else acc + g
    h = jnp.maximum(acc + bc[...], 0.0)
    h = jnp.maximum(_dot(h, wr2[...]) + br2[...], 0.0)
    h = jnp.maximum(_dot(h, wr3[...]) + br3[...], 0.0)
    out_ref[...] = _dot(h, wr4[...]) + br4[...]


def kernel(x, nets, rho_params):
    n, ndim = x.shape
    inter = nets[0][-1][0].shape[1]
    zdim = rho_params[-1][0].shape[1]

    args = []
    for net in nets:
        for (w, b) in net:
            args += [w, b]
    for (w, b) in rho_params:
        args += [w, b]

    blk = 4096
    n_pad = ((n + blk - 1) // blk) * blk
    xp = x if n_pad == n else jnp.pad(x, ((0, n_pad - n), (0, 0)))

    full = lambda a: pl.BlockSpec(a.shape, lambda i: (0,) * a.ndim)
    out = pl.pallas_call(
        functools.partial(_fused_body, ndim=ndim, inter=inter),
        grid=(n_pad // blk,),
        in_specs=[pl.BlockSpec((blk, ndim), lambda i: (i, 0))]
                 + [full(a) for a in args],
        out_specs=pl.BlockSpec((blk, zdim), lambda i: (i, 0)),
        out_shape=jax.ShapeDtypeStruct((n_pad, zdim), jnp.float32),
        scratch_shapes=[pltpu.VMEM((ndim * inter, inter), jnp.float32),
                        pltpu.VMEM((1, inter), jnp.float32)],
    )(xp, *args)
    return out[:n] if n_pad != n else out
